# baseline (speedup 1.0000x reference)
import functools

import jax
import jax.numpy as jnp
from jax import lax
from jax.experimental import pallas as pl
from jax.experimental.pallas import tpu as pltpu
from jax.experimental.pallas import tpu_sc as plsc

_B = 64
_N = 40
_H = 256
_K = 2
_E = _H * _K
_NN = _N * _N
_S = _NN * 2
_NODE_SIZES = (22, 6, 6, 3, 3, 5)
_NT = sum(_NODE_SIZES)
_P = _B * _N
_PS = 512
_PT = _P - _PS
_PTB = 256
_NW = 32
_PPW = _PS // _NW

_TT = (((0,), (0,)), ((), ()))


def _edge_body(comb_ref, wtab_ref, edge_out_ref):
    comb = comb_ref[0]
    iota_e = jax.lax.broadcasted_iota(jnp.int32, (32, _S), 0)
    oh = (iota_e == comb).astype(jnp.float32)
    edge = jax.lax.dot_general(oh, wtab_ref[...], _TT,
                               preferred_element_type=jnp.float32)
    edge_out_ref[...] = edge


def _node_tc_body(nidx_ref, ntab_ref, node_out_ref):
    iota_nt = jax.lax.broadcasted_iota(jnp.int32, (_NT, _PTB), 0)
    cnt_t = jnp.zeros((_NT, _PTB), jnp.float32)
    for t in range(6):
        idx_row = nidx_ref[t:t + 1, :]
        cnt_t += (iota_nt == idx_row).astype(jnp.float32)
    node = jax.lax.dot_general(cnt_t, ntab_ref[...], _TT,
                               preferred_element_type=jnp.float32)
    node_out_ref[...] = node


_node_mesh = plsc.VectorSubcoreMesh(core_axis_name="c", subcore_axis_name="s")


@functools.partial(
    pl.kernel,
    mesh=_node_mesh,
    out_type=jax.ShapeDtypeStruct((_PS, _H), jnp.float32),
    scratch_types=[
        pltpu.VMEM((_PPW * 6,), jnp.int32),
        pltpu.VMEM((_PPW * 6, _H), jnp.float32),
        pltpu.VMEM((_PPW, _H), jnp.float32),
        pltpu.SemaphoreType.DMA,
    ],
)
def _node_sc(idx_hbm, ntab_hbm, node_hbm, idx_v, rows_v, out_v, sem):
    wid = lax.axis_index("s") * 2 + lax.axis_index("c")
    base = wid * _PPW
    pltpu.sync_copy(idx_hbm.at[pl.ds(base * 6, _PPW * 6)], idx_v)
    pltpu.async_copy(ntab_hbm.at[idx_v], rows_v, sem).wait()

    def body(p, carry):
        for j in range(_H // 16):
            sl = pl.ds(j * 16, 16)
            acc = rows_v[p * 6, sl]
            for t in range(1, 6):
                acc = acc + rows_v[p * 6 + t, sl]
            out_v[p, sl] = acc
        return carry

    lax.fori_loop(0, _PPW, body, None)
    pltpu.sync_copy(out_v, node_hbm.at[pl.ds(base, _PPW)])


def kernel(node_inds, adj_mat_inds, init_hydrogens, init_charge,
           init_is_in_ring, init_is_aromatic, init_chirality,
           n_table, e_table, h_table, charge_table, ring_table,
           arom_table, chir_table):
    offs = []
    o = 0
    for sz in _NODE_SIZES:
        offs.append(o)
        o += sz
    idx6 = jnp.stack([node_inds, init_hydrogens, init_charge,
                      init_is_in_ring, init_is_aromatic, init_chirality], 0)
    idx6 = idx6.astype(jnp.int32) + jnp.asarray(offs, jnp.int32)[:, None, None]
    idx_tp = idx6.reshape(6, _P)
    idx_tc = idx_tp[:, :_PT]
    idx_sc = idx_tp[:, _PT:].T.reshape(_PS * 6)
    ntab = jnp.concatenate([n_table, h_table, charge_table, ring_table,
                            arom_table, chir_table], 0)
    ntab48 = jnp.concatenate(
        [ntab, jnp.zeros((48 - _NT, _H), jnp.float32)], 0)

    adj = adj_mat_inds.astype(jnp.int32).reshape(_B, _NN, 1)
    comb = (adj * 4 + jnp.arange(4, dtype=jnp.int32)).reshape(_B * 2, 1, _S)
    wtab = e_table.reshape(7, 2, 128, 2).transpose(0, 1, 3, 2).reshape(28, 128)
    wtab = jnp.concatenate([wtab, jnp.zeros((4, 128), jnp.float32)], 0)

    node_sc = _node_sc(idx_sc, ntab48)

    edge_out = pl.pallas_call(
        _edge_body,
        grid=(_B * 2,),
        in_specs=[
            pl.BlockSpec((1, 1, _S), lambda b: (b, 0, 0)),
            pl.BlockSpec((32, 128), lambda b: (0, 0)),
        ],
        out_specs=pl.BlockSpec((_S, 128), lambda b: (b, 0)),
        out_shape=jax.ShapeDtypeStruct((_B * 2 * _S, 128), jnp.float32),
    )(comb, wtab)

    node_tc = pl.pallas_call(
        _node_tc_body,
        grid=(_PT // _PTB,),
        in_specs=[
            pl.BlockSpec((6, _PTB), lambda s: (0, s)),
            pl.BlockSpec((_NT, _H), lambda s: (0, 0)),
        ],
        out_specs=pl.BlockSpec((_PTB, _H), lambda s: (s, 0)),
        out_shape=jax.ShapeDtypeStruct((_PT, _H), jnp.float32),
    )(idx_tc, ntab)

    node = jnp.concatenate([node_tc, node_sc], 0).reshape(_B, _N, _H)
    edge5 = (edge_out.reshape(_B, _N, _N, 2, _K, 128)
             .transpose(0, 1, 2, 3, 5, 4)
             .reshape(_B, _N, _N, _H, _K))
    return node, edge5


# speedup vs baseline: 1.2639x; 1.2639x over previous
import functools

import jax
import jax.numpy as jnp
from jax import lax
from jax.experimental import pallas as pl
from jax.experimental.pallas import tpu as pltpu
from jax.experimental.pallas import tpu_sc as plsc

_B = 64
_N = 40
_H = 256
_K = 2
_E = _H * _K
_NN = _N * _N
_S = _NN * 8
_NODE_SIZES = (22, 6, 6, 3, 3, 5)
_NT = sum(_NODE_SIZES)
_P = _B * _N
_PS = 512
_PT = _P - _PS
_PTB = 256
_NW = 32
_PPW = _PS // _NW

_TT = (((0,), (0,)), ((), ()))


def _edge_body(comb_ref, wtab_ref, edge_out_ref):
    comb = comb_ref[0]
    iota_e = jax.lax.broadcasted_iota(jnp.int32, (32, _S), 0)
    oh = (iota_e == comb).astype(jnp.float32)
    edge = jax.lax.dot_general(oh, wtab_ref[...], _TT,
                               preferred_element_type=jnp.float32)
    edge_out_ref[...] = edge


def _node_tc_body(nidx_ref, ntab_ref, node_out_ref):
    iota_nt = jax.lax.broadcasted_iota(jnp.int32, (_NT, _PTB), 0)
    cnt_t = jnp.zeros((_NT, _PTB), jnp.float32)
    for t in range(6):
        idx_row = nidx_ref[t:t + 1, :]
        cnt_t += (iota_nt == idx_row).astype(jnp.float32)
    node = jax.lax.dot_general(cnt_t, ntab_ref[...], _TT,
                               preferred_element_type=jnp.float32)
    node_out_ref[...] = node


_node_mesh = plsc.VectorSubcoreMesh(core_axis_name="c", subcore_axis_name="s")


@functools.partial(
    pl.kernel,
    mesh=_node_mesh,
    out_type=jax.ShapeDtypeStruct((_PS, _H), jnp.float32),
    scratch_types=[
        pltpu.VMEM((_PPW * 6,), jnp.int32),
        pltpu.VMEM((_PPW * 6, _H), jnp.float32),
        pltpu.VMEM((_PPW, _H), jnp.float32),
        pltpu.SemaphoreType.DMA,
    ],
)
def _node_sc(idx_hbm, ntab_hbm, node_hbm, idx_v, rows_v, out_v, sem):
    wid = lax.axis_index("s") * 2 + lax.axis_index("c")
    base = wid * _PPW
    pltpu.sync_copy(idx_hbm.at[pl.ds(base * 6, _PPW * 6)], idx_v)
    pltpu.async_copy(ntab_hbm.at[idx_v], rows_v, sem).wait()

    def body(p, carry):
        for j in range(_H // 16):
            sl = pl.ds(j * 16, 16)
            acc = rows_v[p * 6, sl]
            for t in range(1, 6):
                acc = acc + rows_v[p * 6 + t, sl]
            out_v[p, sl] = acc
        return carry

    lax.fori_loop(0, _PPW, body, None)
    pltpu.sync_copy(out_v, node_hbm.at[pl.ds(base, _PPW)])


def kernel(node_inds, adj_mat_inds, init_hydrogens, init_charge,
           init_is_in_ring, init_is_aromatic, init_chirality,
           n_table, e_table, h_table, charge_table, ring_table,
           arom_table, chir_table):
    offs = []
    o = 0
    for sz in _NODE_SIZES:
        offs.append(o)
        o += sz
    idx6 = jnp.stack([node_inds, init_hydrogens, init_charge,
                      init_is_in_ring, init_is_aromatic, init_chirality], 0)
    idx6 = idx6.astype(jnp.int32) + jnp.asarray(offs, jnp.int32)[:, None, None]
    idx_tp = idx6.reshape(6, _P)
    idx_tc = idx_tp[:, :_PT]
    idx_sc = idx_tp[:, _PT:].T.reshape(_PS * 6)
    ntab = jnp.concatenate([n_table, h_table, charge_table, ring_table,
                            arom_table, chir_table], 0)
    ntab48 = jnp.concatenate(
        [ntab, jnp.zeros((48 - _NT, _H), jnp.float32)], 0)

    adj = adj_mat_inds.astype(jnp.int32).reshape(_B, _NN, 1)
    comb = (adj * 4 + jnp.arange(4, dtype=jnp.int32)).reshape(_B // 2, 1, _S)
    wtab = e_table.reshape(7, 2, 128, 2).transpose(0, 1, 3, 2).reshape(28, 128)
    wtab = jnp.concatenate([wtab, jnp.zeros((4, 128), jnp.float32)], 0)

    node_sc = _node_sc(idx_sc, ntab48)

    edge_out = pl.pallas_call(
        _edge_body,
        grid=(_B // 2,),
        in_specs=[
            pl.BlockSpec((1, 1, _S), lambda b: (b, 0, 0)),
            pl.BlockSpec((32, 128), lambda b: (0, 0)),
        ],
        out_specs=pl.BlockSpec((_S, 128), lambda b: (b, 0)),
        out_shape=jax.ShapeDtypeStruct((_B // 2 * _S, 128), jnp.float32),
    )(comb, wtab)

    node_tc = pl.pallas_call(
        _node_tc_body,
        grid=(_PT // _PTB,),
        in_specs=[
            pl.BlockSpec((6, _PTB), lambda s: (0, s)),
            pl.BlockSpec((_NT, _H), lambda s: (0, 0)),
        ],
        out_specs=pl.BlockSpec((_PTB, _H), lambda s: (s, 0)),
        out_shape=jax.ShapeDtypeStruct((_PT, _H), jnp.float32),
    )(idx_tc, ntab)

    node = jnp.concatenate([node_tc, node_sc], 0).reshape(_B, _N, _H)
    edge5 = (edge_out.reshape(_B, _N, _N, 2, _K, 128)
             .transpose(0, 1, 2, 3, 5, 4)
             .reshape(_B, _N, _N, _H, _K))
    return node, edge5


# speedup vs baseline: 1.6976x; 1.3431x over previous
import functools

import jax
import jax.numpy as jnp
from jax import lax
from jax.experimental import pallas as pl
from jax.experimental.pallas import tpu as pltpu
from jax.experimental.pallas import tpu_sc as plsc

_B = 64
_N = 40
_H = 256
_K = 2
_E = _H * _K
_NN = _N * _N
_S = _NN * 4
_NODE_SIZES = (22, 6, 6, 3, 3, 5)
_NT = sum(_NODE_SIZES)
_P = _B * _N
_PS = 512
_PT = _P - _PS
_PTB = 256
_NW = 32
_PPW = _PS // _NW

_TT = (((0,), (0,)), ((), ()))


def _edge_body(comb_ref, wtab_ref, edge_out_ref):
    comb = comb_ref[0]
    iota_e = jax.lax.broadcasted_iota(jnp.int32, (32, _S), 0)
    oh = (iota_e == comb).astype(jnp.float32)
    edge = jax.lax.dot_general(oh, wtab_ref[...], _TT,
                               preferred_element_type=jnp.float32)
    edge_out_ref[...] = edge


def _node_tc_body(nidx_ref, ntab_ref, node_out_ref):
    iota_nt = jax.lax.broadcasted_iota(jnp.int32, (_NT, _PTB), 0)
    cnt_t = jnp.zeros((_NT, _PTB), jnp.float32)
    for t in range(6):
        idx_row = nidx_ref[t:t + 1, :]
        cnt_t += (iota_nt == idx_row).astype(jnp.float32)
    node = jax.lax.dot_general(cnt_t, ntab_ref[...], _TT,
                               preferred_element_type=jnp.float32)
    node_out_ref[...] = node


_node_mesh = plsc.VectorSubcoreMesh(core_axis_name="c", subcore_axis_name="s")


@functools.partial(
    pl.kernel,
    mesh=_node_mesh,
    out_type=jax.ShapeDtypeStruct((_PS, _H), jnp.float32),
    scratch_types=[
        pltpu.VMEM((_PPW * 6,), jnp.int32),
        pltpu.VMEM((_PPW * 6, _H), jnp.float32),
        pltpu.VMEM((_PPW, _H), jnp.float32),
        pltpu.SemaphoreType.DMA,
    ],
)
def _node_sc(idx_hbm, ntab_hbm, node_hbm, idx_v, rows_v, out_v, sem):
    wid = lax.axis_index("s") * 2 + lax.axis_index("c")
    base = wid * _PPW
    pltpu.sync_copy(idx_hbm.at[pl.ds(base * 6, _PPW * 6)], idx_v)
    pltpu.async_copy(ntab_hbm.at[idx_v], rows_v, sem).wait()

    def body(p, carry):
        for j in range(_H // 16):
            sl = pl.ds(j * 16, 16)
            acc = rows_v[p * 6, sl]
            for t in range(1, 6):
                acc = acc + rows_v[p * 6 + t, sl]
            out_v[p, sl] = acc
        return carry

    lax.fori_loop(0, _PPW, body, None)
    pltpu.sync_copy(out_v, node_hbm.at[pl.ds(base, _PPW)])


def kernel(node_inds, adj_mat_inds, init_hydrogens, init_charge,
           init_is_in_ring, init_is_aromatic, init_chirality,
           n_table, e_table, h_table, charge_table, ring_table,
           arom_table, chir_table):
    offs = []
    o = 0
    for sz in _NODE_SIZES:
        offs.append(o)
        o += sz
    idx6 = jnp.stack([node_inds, init_hydrogens, init_charge,
                      init_is_in_ring, init_is_aromatic, init_chirality], 0)
    idx6 = idx6.astype(jnp.int32) + jnp.asarray(offs, jnp.int32)[:, None, None]
    idx_tp = idx6.reshape(6, _P)
    idx_tc = idx_tp[:, :_PT]
    idx_sc = idx_tp[:, _PT:].T.reshape(_PS * 6)
    ntab = jnp.concatenate([n_table, h_table, charge_table, ring_table,
                            arom_table, chir_table], 0)
    ntab48 = jnp.concatenate(
        [ntab, jnp.zeros((48 - _NT, _H), jnp.float32)], 0)

    adj = adj_mat_inds.astype(jnp.int32).reshape(_B, _NN, 1)
    comb = (adj * 4 + jnp.arange(4, dtype=jnp.int32)).reshape(_B, 1, _S)
    wtab = e_table.reshape(7, 2, 128, 2).transpose(0, 1, 3, 2).reshape(28, 128)
    wtab = jnp.concatenate([wtab, jnp.zeros((4, 128), jnp.float32)], 0)

    node_sc = _node_sc(idx_sc, ntab48)

    edge_out = pl.pallas_call(
        _edge_body,
        grid=(_B,),
        in_specs=[
            pl.BlockSpec((1, 1, _S), lambda b: (b, 0, 0)),
            pl.BlockSpec((32, 128), lambda b: (0, 0)),
        ],
        out_specs=pl.BlockSpec((_S, 128), lambda b: (b, 0)),
        out_shape=jax.ShapeDtypeStruct((_B * _S, 128), jnp.float32),
    )(comb, wtab)

    node_tc = pl.pallas_call(
        _node_tc_body,
        grid=(_PT // _PTB,),
        in_specs=[
            pl.BlockSpec((6, _PTB), lambda s: (0, s)),
            pl.BlockSpec((_NT, _H), lambda s: (0, 0)),
        ],
        out_specs=pl.BlockSpec((_PTB, _H), lambda s: (s, 0)),
        out_shape=jax.ShapeDtypeStruct((_PT, _H), jnp.float32),
    )(idx_tc, ntab)

    node = jnp.concatenate([node_tc, node_sc], 0).reshape(_B, _N, _H)
    edge5 = (edge_out.reshape(_B, _N, _N, 2, _K, 128)
             .transpose(0, 1, 2, 3, 5, 4)
             .reshape(_B, _N, _N, _H, _K))
    return node, edge5


# speedup vs baseline: 1.7570x; 1.0350x over previous
import functools

import jax
import jax.numpy as jnp
from jax import lax
from jax.experimental import pallas as pl
from jax.experimental.pallas import tpu as pltpu
from jax.experimental.pallas import tpu_sc as plsc

_B = 64
_N = 40
_H = 256
_K = 2
_E = _H * _K
_NN = _N * _N
_S = _NN * 4
_NODE_SIZES = (22, 6, 6, 3, 3, 5)
_NT = sum(_NODE_SIZES)
_P = _B * _N
_PS = 512
_PT = _P - _PS
_PTB = 32
_NW = 32
_PPW = _PS // _NW

_TT = (((0,), (0,)), ((), ()))


def _edge_body(comb_ref, wtab_ref, nidx_ref, ntab_ref,
               edge_out_ref, node_out_ref):
    comb = comb_ref[0]
    iota_e = jax.lax.broadcasted_iota(jnp.int32, (32, _S), 0)
    oh = (iota_e == comb).astype(jnp.float32)
    edge = jax.lax.dot_general(oh, wtab_ref[...], _TT,
                               preferred_element_type=jnp.float32)
    edge_out_ref[...] = edge

    iota_nt = jax.lax.broadcasted_iota(jnp.int32, (_NT, _PTB), 0)
    cnt_t = jnp.zeros((_NT, _PTB), jnp.float32)
    for t in range(6):
        idx_row = nidx_ref[0, t:t + 1, :]
        cnt_t += (iota_nt == idx_row).astype(jnp.float32)
    node = jax.lax.dot_general(cnt_t, ntab_ref[...], _TT,
                               preferred_element_type=jnp.float32)
    node_out_ref[...] = node


_node_mesh = plsc.VectorSubcoreMesh(core_axis_name="c", subcore_axis_name="s")


@functools.partial(
    pl.kernel,
    mesh=_node_mesh,
    out_type=jax.ShapeDtypeStruct((_PS, _H), jnp.float32),
    scratch_types=[
        pltpu.VMEM((_PPW * 6,), jnp.int32),
        pltpu.VMEM((_PPW * 6, _H), jnp.float32),
        pltpu.VMEM((_PPW, _H), jnp.float32),
        pltpu.SemaphoreType.DMA,
    ],
)
def _node_sc(idx_hbm, ntab_hbm, node_hbm, idx_v, rows_v, out_v, sem):
    wid = lax.axis_index("s") * 2 + lax.axis_index("c")
    base = wid * _PPW
    pltpu.sync_copy(idx_hbm.at[pl.ds(base * 6, _PPW * 6)], idx_v)
    pltpu.async_copy(ntab_hbm.at[idx_v], rows_v, sem).wait()

    def body(p, carry):
        for j in range(_H // 16):
            sl = pl.ds(j * 16, 16)
            acc = rows_v[p * 6, sl]
            for t in range(1, 6):
                acc = acc + rows_v[p * 6 + t, sl]
            out_v[p, sl] = acc
        return carry

    lax.fori_loop(0, _PPW, body, None)
    pltpu.sync_copy(out_v, node_hbm.at[pl.ds(base, _PPW)])


def kernel(node_inds, adj_mat_inds, init_hydrogens, init_charge,
           init_is_in_ring, init_is_aromatic, init_chirality,
           n_table, e_table, h_table, charge_table, ring_table,
           arom_table, chir_table):
    offs = []
    o = 0
    for sz in _NODE_SIZES:
        offs.append(o)
        o += sz
    idx6 = jnp.stack([node_inds, init_hydrogens, init_charge,
                      init_is_in_ring, init_is_aromatic, init_chirality], 0)
    idx6 = idx6.astype(jnp.int32) + jnp.asarray(offs, jnp.int32)[:, None, None]
    idx_tp = idx6.reshape(6, _P)
    idx_tc = (idx_tp[:, :_PT].reshape(6, _B, _PTB)
              .transpose(1, 0, 2))
    idx_sc = idx_tp[:, _PT:].T.reshape(_PS * 6)
    ntab = jnp.concatenate([n_table, h_table, charge_table, ring_table,
                            arom_table, chir_table], 0)
    ntab48 = jnp.concatenate(
        [ntab, jnp.zeros((48 - _NT, _H), jnp.float32)], 0)

    adj = adj_mat_inds.astype(jnp.int32).reshape(_B, _NN, 1)
    comb = (adj * 4 + jnp.arange(4, dtype=jnp.int32)).reshape(_B, 1, _S)
    wtab = e_table.reshape(7, 2, 128, 2).transpose(0, 1, 3, 2).reshape(28, 128)
    wtab = jnp.concatenate([wtab, jnp.zeros((4, 128), jnp.float32)], 0)

    node_sc = _node_sc(idx_sc, ntab48)

    edge_out, node_tc = pl.pallas_call(
        _edge_body,
        grid=(_B,),
        in_specs=[
            pl.BlockSpec((1, 1, _S), lambda b: (b, 0, 0)),
            pl.BlockSpec((32, 128), lambda b: (0, 0)),
            pl.BlockSpec((1, 6, _PTB), lambda b: (b, 0, 0)),
            pl.BlockSpec((_NT, _H), lambda b: (0, 0)),
        ],
        out_specs=[
            pl.BlockSpec((_S, 128), lambda b: (b, 0)),
            pl.BlockSpec((_PTB, _H), lambda b: (b, 0)),
        ],
        out_shape=[
            jax.ShapeDtypeStruct((_B * _S, 128), jnp.float32),
            jax.ShapeDtypeStruct((_PT, _H), jnp.float32),
        ],
    )(comb, wtab, idx_tc, ntab)

    node = jnp.concatenate([node_tc, node_sc], 0).reshape(_B, _N, _H)
    edge5 = (edge_out.reshape(_B, _N, _N, 2, _K, 128)
             .transpose(0, 1, 2, 3, 5, 4)
             .reshape(_B, _N, _N, _H, _K))
    return node, edge5
